# baseline (speedup 1.0000x reference)
import jax
import jax.numpy as jnp
from jax import lax
from jax.experimental import pallas as pl
from jax.experimental.pallas import tpu as pltpu, tpu_sc as plsc

B, N, D = 1024, 100, 128
T = 2 * N
NC, NS, L = 2, 16, 16
NW = NC * NS
BPW = B // NW
NJ = D // L

_CHUNK_OFFS = (0, 16, 32, 48, 64, 80, 84)


def _body(tr_hbm, im_hbm, vt_hbm, wv_hbm, ctab_hbm, out_hbm,
          t_v, m_v, ft_v, wv_v, ctab_v, vm, shared, sem0, sem1):
    cid = lax.axis_index("c")
    sid = lax.axis_index("s")
    wid = sid * NC + cid
    base = wid * BPW

    pltpu.sync_copy(tr_hbm.at[pl.ds(base, BPW)], t_v)
    pltpu.sync_copy(im_hbm.at[pl.ds(base, BPW)], m_v)
    pltpu.sync_copy(vt_hbm, ft_v)
    pltpu.sync_copy(wv_hbm, wv_v)
    pltpu.sync_copy(ctab_hbm, ctab_v)

    def fill(n, c):
        for j in range(NJ):
            vm[2 * n, pl.ds(j * L, L)] = ft_v[n, pl.ds(j * L, L)]
        return c

    lax.fori_loop(0, N, fill, 0)

    wv_r = [wv_v[pl.ds(j * L, L)] for j in range(NJ)]
    c0_r = [ctab_v[0, pl.ds(j * L, L)] for j in range(NJ)]
    c1_r = [ctab_v[1, pl.ds(j * L, L)] for j in range(NJ)]

    def compute_odd(bl):
        for off in _CHUNK_OFFS:
            tv = t_v[bl, pl.ds(off, L)]
            miv = m_v[bl, pl.ds(off, L)].astype(jnp.int32)
            for l in range(L):
                n = off + l
                t_s = tv[l]
                pick = miv[l] != 0
                for j in range(NJ):
                    cj = jnp.where(pick, c1_r[j], c0_r[j])
                    vm[2 * n + 1, pl.ds(j * L, L)] = t_s * wv_r[j] + cj

    def stage_and_fire(k, sem, bl):
        pltpu.sync_copy(vm, shared.at[sid, k])
        pltpu.async_copy(shared.at[sid, k], out_hbm.at[base + bl], sem)

    compute_odd(0)
    stage_and_fire(0, sem0, 0)
    compute_odd(1)
    stage_and_fire(1, sem1, 1)

    def per_pair(p, c):
        bl0 = 2 + 2 * p
        for k in range(2):
            sem = sem0 if k == 0 else sem1
            bl = bl0 + k
            pltpu.make_async_copy(shared.at[sid, k], out_hbm.at[base], sem).wait()
            compute_odd(bl)
            stage_and_fire(k, sem, bl)
        return c

    lax.fori_loop(0, (BPW - 2) // 2, per_pair, 0)
    pltpu.make_async_copy(shared.at[sid, 0], out_hbm.at[base], sem0).wait()
    pltpu.make_async_copy(shared.at[sid, 1], out_hbm.at[base], sem1).wait()


def kernel(base_samples, int_samples, target_row, int_mask, var_table, Wv, bv, type_table):
    wv = Wv[0]
    ctab = type_table + bv[None, :]
    mesh = plsc.VectorSubcoreMesh(core_axis_name="c", subcore_axis_name="s")
    k = pl.kernel(
        _body,
        out_type=jax.ShapeDtypeStruct((B, T, D), jnp.float32),
        mesh=mesh,
        scratch_types=[
            pltpu.VMEM((BPW, N), jnp.float32),
            pltpu.VMEM((BPW, N), jnp.float32),
            pltpu.VMEM((N, D), jnp.float32),
            pltpu.VMEM((D,), jnp.float32),
            pltpu.VMEM((2, D), jnp.float32),
            pltpu.VMEM((T, D), jnp.float32),
            pltpu.MemorySpace.VMEM_SHARED((NS, 2, T, D), jnp.float32),
            pltpu.SemaphoreType.DMA,
            pltpu.SemaphoreType.DMA,
        ],
    )
    return k(target_row, int_mask, var_table[:N], wv, ctab)


# speedup vs baseline: 1.3748x; 1.3748x over previous
import jax
import jax.numpy as jnp
from jax import lax
from jax.experimental import pallas as pl
from jax.experimental.pallas import tpu as pltpu, tpu_sc as plsc

B, N, D = 1024, 100, 128
T = 2 * N
NC, NS, L = 2, 16, 16
NW = NC * NS
BPW = B // NW
NJ = D // L

_CHUNK_OFFS = (0, 16, 32, 48, 64, 80, 84)


def _body(tr_hbm, im_hbm, vt_hbm, wv_hbm, ctab_hbm, out_hbm,
          t_v, m_v, ft_v, wv_v, ctab_v, vm, sem_s):
    wid = lax.axis_index("s") * NC + lax.axis_index("c")
    base = wid * BPW

    h1 = pltpu.async_copy(tr_hbm.at[pl.ds(base, BPW)], t_v, sem_s)
    h2 = pltpu.async_copy(im_hbm.at[pl.ds(base, BPW)], m_v, sem_s)
    h3 = pltpu.async_copy(vt_hbm, ft_v, sem_s)
    h4 = pltpu.async_copy(wv_hbm, wv_v, sem_s)
    h5 = pltpu.async_copy(ctab_hbm, ctab_v, sem_s)
    h1.wait()
    h2.wait()
    h3.wait()
    h4.wait()
    h5.wait()

    def fill(n, c):
        for j in range(NJ):
            vm[2 * n, pl.ds(j * L, L)] = ft_v[n, pl.ds(j * L, L)]
        return c

    lax.fori_loop(0, N, fill, 0)

    wv_r = [wv_v[pl.ds(j * L, L)] for j in range(NJ)]
    c0_r = [ctab_v[0, pl.ds(j * L, L)] for j in range(NJ)]
    c1_r = [ctab_v[1, pl.ds(j * L, L)] for j in range(NJ)]

    def per_batch(bl, c):
        for off in _CHUNK_OFFS:
            tv = t_v[bl, pl.ds(off, L)]
            miv = m_v[bl, pl.ds(off, L)].astype(jnp.int32)
            for l in range(L):
                n = off + l
                t_s = tv[l]
                pick = miv[l] != 0
                for j in range(NJ):
                    cj = jnp.where(pick, c1_r[j], c0_r[j])
                    vm[2 * n + 1, pl.ds(j * L, L)] = t_s * wv_r[j] + cj
        pltpu.sync_copy(vm, out_hbm.at[base + bl])
        return c

    lax.fori_loop(0, BPW, per_batch, 0)


def kernel(base_samples, int_samples, target_row, int_mask, var_table, Wv, bv, type_table):
    wv = Wv[0]
    ctab = type_table + bv[None, :]
    mesh = plsc.VectorSubcoreMesh(core_axis_name="c", subcore_axis_name="s")
    k = pl.kernel(
        _body,
        out_type=jax.ShapeDtypeStruct((B, T, D), jnp.float32),
        mesh=mesh,
        scratch_types=[
            pltpu.VMEM((BPW, N), jnp.float32),
            pltpu.VMEM((BPW, N), jnp.float32),
            pltpu.VMEM((N, D), jnp.float32),
            pltpu.VMEM((D,), jnp.float32),
            pltpu.VMEM((2, D), jnp.float32),
            pltpu.VMEM((T, D), jnp.float32),
            pltpu.SemaphoreType.DMA,
        ],
    )
    return k(target_row, int_mask, var_table[:N], wv, ctab)
